# Initial kernel scaffold; baseline (speedup 1.0000x reference)
import jax
import jax.numpy as jnp
from jax.experimental import pallas as pl


def kernel(x_nchw, w1, b1, w2, rand_mask):
    raise NotImplementedError("write your pallas kernel here")


# speedup vs baseline: 1.7545x; 1.7545x over previous
import functools

import jax
import jax.numpy as jnp
from jax.experimental import pallas as pl
from jax.experimental.pallas import tpu as pltpu

_ALPHA = 3
_ALIVE_THRESHOLD = 0.1
_G = 8


def _nca_kernel(x_ref, w1bd_ref, b1_ref, w2bd_ref, sel_ref, mask_ref, o_ref,
                *, n_channels, height, width):
    C, H, W = n_channels, height, width
    HW = H * W
    x = x_ref[...]
    lane = jax.lax.broadcasted_iota(jnp.int32, x.shape, 1)
    wcol = lane % W
    hrow = lane // W
    neg_inf = jnp.float32(-jnp.inf)

    def sh_h(y, oy):
        return pltpu.roll(y, (-oy * W) % HW, axis=1)

    def sh_w(y, ox):
        main = pltpu.roll(y, (-ox) % HW, axis=1)
        wrap = pltpu.roll(y, (-ox + (W if ox > 0 else -W)) % HW, axis=1)
        edge = (wcol == W - 1) if ox > 0 else (wcol == 0)
        return jnp.where(edge, wrap, main)

    sm_h = sh_h(x, -1) + 2.0 * x + sh_h(x, 1)
    grad_x = sh_w(sm_h, 1) - sh_w(sm_h, -1)
    sm_w = sh_w(x, -1) + 2.0 * x + sh_w(x, 1)
    grad_y = sh_h(sm_w, -1) - sh_h(sm_w, 1)

    percept = jnp.concatenate([x, grad_x, grad_y], axis=0).astype(jnp.bfloat16)
    h = jnp.dot(w1bd_ref[...], percept,
                preferred_element_type=jnp.float32) + b1_ref[...]
    h = jnp.maximum(h, 0.0).astype(jnp.bfloat16)
    ds = jnp.dot(w2bd_ref[...], h,
                 preferred_element_type=jnp.float32)

    raw = x + ds * mask_ref[...]

    def pool_w(y, ox):
        shifted = sh_w(y, ox)
        edge = (wcol >= W - ox) if ox > 0 else (wcol < -ox)
        return jnp.where(edge, neg_inf, shifted)

    def pool_h(y, oy):
        shifted = sh_h(y, oy)
        edge = (hrow >= H - oy) if oy > 0 else (hrow < -oy)
        return jnp.where(edge, neg_inf, shifted)

    pooled_w = jnp.maximum(raw, jnp.maximum(pool_w(raw, -1), pool_w(raw, 1)))
    pooled = jnp.maximum(pooled_w,
                         jnp.maximum(pool_h(pooled_w, -1), pool_h(pooled_w, 1)))
    alive = (pooled > _ALIVE_THRESHOLD).astype(jnp.bfloat16)
    alive_b = jnp.dot(sel_ref[...], alive, preferred_element_type=jnp.float32)

    o_ref[...] = raw * alive_b


def _nca_step(x_flat, w1bd, b1col, w2bd, sel, mask_flat, C, H, W):
    BC, HW = x_flat.shape
    rows = _G * C
    body = functools.partial(_nca_kernel, n_channels=C, height=H, width=W)
    return pl.pallas_call(
        body,
        grid=(BC // rows,),
        out_shape=jax.ShapeDtypeStruct((BC, HW), jnp.float32),
        in_specs=[
            pl.BlockSpec((rows, HW), lambda b: (b, 0)),
            pl.BlockSpec(w1bd.shape, lambda b: (0, 0)),
            pl.BlockSpec(b1col.shape, lambda b: (0, 0)),
            pl.BlockSpec(w2bd.shape, lambda b: (0, 0)),
            pl.BlockSpec(sel.shape, lambda b: (0, 0)),
            pl.BlockSpec(mask_flat.shape, lambda b: (0, 0)),
        ],
        out_specs=pl.BlockSpec((rows, HW), lambda b: (b, 0)),
        compiler_params=pltpu.CompilerParams(
            dimension_semantics=("parallel",)),
    )(x_flat, w1bd, b1col, w2bd, sel, mask_flat)


def kernel(x_nchw, w1, b1, w2, rand_mask):
    B, C, H, W = x_nchw.shape
    hidden = w1.shape[1]
    x_flat = x_nchw.reshape(B * C, H * W)

    eye_g = jnp.eye(_G, dtype=jnp.float32)
    w1bd = jnp.concatenate(
        [jnp.kron(eye_g, w1[t * C:(t + 1) * C].T) for t in range(3)],
        axis=1).astype(jnp.bfloat16)
    w2bd = jnp.kron(eye_g, w2.T).astype(jnp.bfloat16)
    b1col = jnp.tile(b1, _G)[:, None]
    sel8 = (jnp.arange(C)[None, :] == _ALPHA).astype(jnp.float32)
    sel = jnp.kron(eye_g, jnp.broadcast_to(sel8, (C, C))).astype(jnp.bfloat16)
    mask_flat = rand_mask.reshape(1, H * W)

    out = _nca_step(x_flat, w1bd, b1col, w2bd, sel, mask_flat, C, H, W)
    return out.reshape(B, C, H, W)


# speedup vs baseline: 1.8707x; 1.0662x over previous
import functools

import jax
import jax.numpy as jnp
import numpy as np
from jax.experimental import pallas as pl
from jax.experimental.pallas import tpu as pltpu

_ALPHA = 3
_ALIVE_THRESHOLD = 0.1
_G = 8


def _nca_kernel(x_ref, gx_ref, gy_ref, w1bd_ref, b1_ref, w2bd_ref,
                selx_ref, selb_ref, mask_ref, o_ref, *, height, width):
    H, W = height, width
    HW = H * W
    x = x_ref[...]
    xb = x.astype(jnp.bfloat16)

    grad_x = jnp.dot(xb, gx_ref[...],
                     preferred_element_type=jnp.float32).astype(jnp.bfloat16)
    grad_y = jnp.dot(xb, gy_ref[...],
                     preferred_element_type=jnp.float32).astype(jnp.bfloat16)

    percept = jnp.concatenate([xb, grad_x, grad_y], axis=0)
    h = jnp.dot(w1bd_ref[...], percept,
                preferred_element_type=jnp.float32) + b1_ref[...]
    h = jnp.maximum(h, 0.0).astype(jnp.bfloat16)
    ds = jnp.dot(w2bd_ref[...], h,
                 preferred_element_type=jnp.float32)

    raw = x + ds * mask_ref[...]

    alpha = jnp.dot(selx_ref[...], raw.astype(jnp.bfloat16),
                    preferred_element_type=jnp.float32)
    lane = jax.lax.broadcasted_iota(jnp.int32, alpha.shape, 1)
    wcol = lane % W
    hrow = lane // W
    neg_inf = jnp.float32(-jnp.inf)
    left = jnp.where(wcol >= 1, pltpu.roll(alpha, 1, axis=1), neg_inf)
    right = jnp.where(wcol <= W - 2, pltpu.roll(alpha, HW - 1, axis=1), neg_inf)
    pw = jnp.maximum(alpha, jnp.maximum(left, right))
    up = jnp.where(hrow >= 1, pltpu.roll(pw, W, axis=1), neg_inf)
    down = jnp.where(hrow <= H - 2, pltpu.roll(pw, HW - W, axis=1), neg_inf)
    pooled = jnp.maximum(pw, jnp.maximum(up, down))
    alive = (pooled > _ALIVE_THRESHOLD).astype(jnp.bfloat16)
    alive_b = jnp.dot(selb_ref[...], alive,
                      preferred_element_type=jnp.float32)

    o_ref[...] = raw * alive_b


def _sobel_ops(H, W):
    HW = H * W
    idx = np.arange(HW)
    h, w = idx // W, idx % W
    gx = np.zeros((HW, HW), np.float32)
    gy = np.zeros((HW, HW), np.float32)
    for d, a in ((-1, 1.0), (0, 2.0), (1, 1.0)):
        for s, sign in ((1, 1.0), (-1, -1.0)):
            src = ((h + d) % H) * W + (w + s) % W
            np.add.at(gx, (src, idx), sign * a)
            src = ((h - s) % H) * W + (w + d) % W
            np.add.at(gy, (src, idx), sign * a)
    return gx, gy


def _nca_step(x_flat, gx, gy, w1bd, b1col, w2bd, selx, selb, mask_flat, C, H, W):
    BC, HW = x_flat.shape
    rows = _G * C
    body = functools.partial(_nca_kernel, height=H, width=W)
    return pl.pallas_call(
        body,
        grid=(BC // rows,),
        out_shape=jax.ShapeDtypeStruct((BC, HW), jnp.float32),
        in_specs=[
            pl.BlockSpec((rows, HW), lambda b: (b, 0)),
            pl.BlockSpec(gx.shape, lambda b: (0, 0)),
            pl.BlockSpec(gy.shape, lambda b: (0, 0)),
            pl.BlockSpec(w1bd.shape, lambda b: (0, 0)),
            pl.BlockSpec(b1col.shape, lambda b: (0, 0)),
            pl.BlockSpec(w2bd.shape, lambda b: (0, 0)),
            pl.BlockSpec(selx.shape, lambda b: (0, 0)),
            pl.BlockSpec(selb.shape, lambda b: (0, 0)),
            pl.BlockSpec(mask_flat.shape, lambda b: (0, 0)),
        ],
        out_specs=pl.BlockSpec((rows, HW), lambda b: (b, 0)),
        compiler_params=pltpu.CompilerParams(
            dimension_semantics=("parallel",)),
    )(x_flat, gx, gy, w1bd, b1col, w2bd, selx, selb, mask_flat)


def kernel(x_nchw, w1, b1, w2, rand_mask):
    B, C, H, W = x_nchw.shape
    x_flat = x_nchw.reshape(B * C, H * W)

    gx_np, gy_np = _sobel_ops(H, W)
    gx = jnp.asarray(gx_np, jnp.bfloat16)
    gy = jnp.asarray(gy_np, jnp.bfloat16)

    eye_g = jnp.eye(_G, dtype=jnp.float32)
    w1bd = jnp.concatenate(
        [jnp.kron(eye_g, w1[t * C:(t + 1) * C].T) for t in range(3)],
        axis=1).astype(jnp.bfloat16)
    w2bd = jnp.kron(eye_g, w2.T).astype(jnp.bfloat16)
    b1col = jnp.tile(b1, _G)[:, None]

    selx_np = np.zeros((_G, _G * C), np.float32)
    selx_np[np.arange(_G), np.arange(_G) * C + _ALPHA] = 1.0
    selb_np = np.zeros((_G * C, _G), np.float32)
    selb_np[np.arange(_G * C), np.arange(_G * C) // C] = 1.0
    selx = jnp.asarray(selx_np, jnp.bfloat16)
    selb = jnp.asarray(selb_np, jnp.bfloat16)
    mask_flat = rand_mask.reshape(1, H * W)

    out = _nca_step(x_flat, gx, gy, w1bd, b1col, w2bd, selx, selb,
                    mask_flat, C, H, W)
    return out.reshape(B, C, H, W)


# speedup vs baseline: 2.5220x; 1.3482x over previous
import functools

import jax
import jax.numpy as jnp
import numpy as np
from jax.experimental import pallas as pl
from jax.experimental.pallas import tpu as pltpu

_ALPHA = 3
_ALIVE_THRESHOLD = 0.1
_G = 8


def _nca_kernel(x_ref, gx_ref, gy_ref, w1bd_ref, b1_ref, w2bd_ref,
                selx_ref, selb_ref, mask_ref, o_ref, *, height, width):
    H, W = height, width
    HW = H * W
    rows = x_ref.shape[0]
    x = x_ref[...].reshape(rows, HW)
    xb = x.astype(jnp.bfloat16)

    grad_x = jnp.dot(xb, gx_ref[...],
                     preferred_element_type=jnp.float32).astype(jnp.bfloat16)
    grad_y = jnp.dot(xb, gy_ref[...],
                     preferred_element_type=jnp.float32).astype(jnp.bfloat16)

    percept = jnp.concatenate([xb, grad_x, grad_y], axis=0)
    h = jnp.dot(w1bd_ref[...], percept,
                preferred_element_type=jnp.float32) + b1_ref[...]
    h = jnp.maximum(h, 0.0).astype(jnp.bfloat16)
    ds = jnp.dot(w2bd_ref[...], h,
                 preferred_element_type=jnp.float32)

    raw = x + ds * mask_ref[...]

    alpha = jnp.dot(selx_ref[...], raw.astype(jnp.bfloat16),
                    preferred_element_type=jnp.float32)
    lane = jax.lax.broadcasted_iota(jnp.int32, alpha.shape, 1)
    wcol = lane % W
    hrow = lane // W
    neg_inf = jnp.float32(-jnp.inf)
    left = jnp.where(wcol >= 1, pltpu.roll(alpha, 1, axis=1), neg_inf)
    right = jnp.where(wcol <= W - 2, pltpu.roll(alpha, HW - 1, axis=1), neg_inf)
    pw = jnp.maximum(alpha, jnp.maximum(left, right))
    up = jnp.where(hrow >= 1, pltpu.roll(pw, W, axis=1), neg_inf)
    down = jnp.where(hrow <= H - 2, pltpu.roll(pw, HW - W, axis=1), neg_inf)
    pooled = jnp.maximum(pw, jnp.maximum(up, down))
    alive = (pooled > _ALIVE_THRESHOLD).astype(jnp.bfloat16)
    alive_b = jnp.dot(selb_ref[...], alive,
                      preferred_element_type=jnp.float32)

    o_ref[...] = (raw * alive_b).reshape(rows, H, W)


def _sobel_ops(H, W):
    HW = H * W
    idx = np.arange(HW)
    h, w = idx // W, idx % W
    gx = np.zeros((HW, HW), np.float32)
    gy = np.zeros((HW, HW), np.float32)
    for d, a in ((-1, 1.0), (0, 2.0), (1, 1.0)):
        for s, sign in ((1, 1.0), (-1, -1.0)):
            src = ((h + d) % H) * W + (w + s) % W
            np.add.at(gx, (src, idx), sign * a)
            src = ((h - s) % H) * W + (w + d) % W
            np.add.at(gy, (src, idx), sign * a)
    return gx, gy


def _nca_step(x3, gx, gy, w1bd, b1col, w2bd, selx, selb, mask_flat, C, H, W):
    BC = x3.shape[0]
    rows = _G * C
    body = functools.partial(_nca_kernel, height=H, width=W)
    return pl.pallas_call(
        body,
        grid=(BC // rows,),
        out_shape=jax.ShapeDtypeStruct((BC, H, W), jnp.float32),
        in_specs=[
            pl.BlockSpec((rows, H, W), lambda b: (b, 0, 0)),
            pl.BlockSpec(gx.shape, lambda b: (0, 0)),
            pl.BlockSpec(gy.shape, lambda b: (0, 0)),
            pl.BlockSpec(w1bd.shape, lambda b: (0, 0)),
            pl.BlockSpec(b1col.shape, lambda b: (0, 0)),
            pl.BlockSpec(w2bd.shape, lambda b: (0, 0)),
            pl.BlockSpec(selx.shape, lambda b: (0, 0)),
            pl.BlockSpec(selb.shape, lambda b: (0, 0)),
            pl.BlockSpec(mask_flat.shape, lambda b: (0, 0)),
        ],
        out_specs=pl.BlockSpec((rows, H, W), lambda b: (b, 0, 0)),
        compiler_params=pltpu.CompilerParams(
            dimension_semantics=("parallel",)),
    )(x3, gx, gy, w1bd, b1col, w2bd, selx, selb, mask_flat)


def kernel(x_nchw, w1, b1, w2, rand_mask):
    B, C, H, W = x_nchw.shape
    x3 = x_nchw.reshape(B * C, H, W)

    gx_np, gy_np = _sobel_ops(H, W)
    gx = jnp.asarray(gx_np, jnp.bfloat16)
    gy = jnp.asarray(gy_np, jnp.bfloat16)

    eye_g = jnp.eye(_G, dtype=jnp.float32)
    w1bd = jnp.concatenate(
        [jnp.kron(eye_g, w1[t * C:(t + 1) * C].T) for t in range(3)],
        axis=1).astype(jnp.bfloat16)
    w2bd = jnp.kron(eye_g, w2.T).astype(jnp.bfloat16)
    b1col = jnp.tile(b1, _G)[:, None]

    selx_np = np.zeros((_G, _G * C), np.float32)
    selx_np[np.arange(_G), np.arange(_G) * C + _ALPHA] = 1.0
    selb_np = np.zeros((_G * C, _G), np.float32)
    selb_np[np.arange(_G * C), np.arange(_G * C) // C] = 1.0
    selx = jnp.asarray(selx_np, jnp.bfloat16)
    selb = jnp.asarray(selb_np, jnp.bfloat16)
    mask_flat = rand_mask.reshape(1, H * W)

    out = _nca_step(x3, gx, gy, w1bd, b1col, w2bd, selx, selb,
                    mask_flat, C, H, W)
    return out.reshape(B, C, H, W)
